# baseline (speedup 1.0000x reference)
import functools

import jax
import jax.numpy as jnp
from jax import lax
from jax.experimental import pallas as pl
from jax.experimental.pallas import tpu as pltpu
from jax.experimental.pallas import tpu_sc as plsc

_EPS = 1e-5
_NC, _NS = 2, 16
_CH = 128
_ROWS = 2000


def _ln(y, g, b):
    lat = y.shape[-1]
    ones = jnp.full((lat, 1), 1.0 / lat, jnp.float32)
    mu = jnp.dot(y, ones, preferred_element_type=jnp.float32)
    ms = jnp.dot(y * y, ones, preferred_element_type=jnp.float32)
    var = ms - mu * mu
    return (y - mu) * lax.rsqrt(var + _EPS) * g + b


def _dot(a, b):
    return jnp.dot(a, b, preferred_element_type=jnp.float32)


def _rep(a):
    return pl.BlockSpec(a.shape, lambda i: (0,) * a.ndim)


def _rows(a, r):
    return pl.BlockSpec((r, a.shape[1]), lambda i: (i, 0))


def _enc_node(x, we1, be1, we2, be2, g, b, w1ab, b1ab):
    n = x.shape[0]
    lat = we2.shape[1]
    r = _ROWS

    def body(x_r, we1_r, be1_r, we2_r, be2_r, g_r, b_r, wab_r, b1ab_r,
             h_r, hab_r):
        xx = x_r[...]
        mid = jnp.maximum(_dot(xx, we1_r[...]) + be1_r[...], 0.0)
        h = _ln(_dot(mid, we2_r[...]) + be2_r[...], g_r[...], b_r[...])
        h_r[...] = h
        hab_r[...] = _dot(h, wab_r[...]) + b1ab_r[...]

    return pl.pallas_call(
        body, grid=(n // r,),
        in_specs=[_rows(x, r)] + [_rep(a) for a in (we1, be1, we2, be2, g, b,
                                                    w1ab, b1ab)],
        out_specs=[pl.BlockSpec((r, lat), lambda i: (i, 0)),
                   pl.BlockSpec((r, 2 * lat), lambda i: (i, 0))],
        out_shape=[jax.ShapeDtypeStruct((n, lat), jnp.float32),
                   jax.ShapeDtypeStruct((n, 2 * lat), jnp.float32)],
    )(x, we1, be1, we2, be2, g, b, w1ab, b1ab)


def _enc_edge(ef, we1, be1, we2, be2, g, b):
    n = ef.shape[0]
    lat = we2.shape[1]
    r = _ROWS

    def body(ef_r, we1_r, be1_r, we2_r, be2_r, g_r, b_r, e_r):
        mid = jnp.maximum(_dot(ef_r[...], we1_r[...]) + be1_r[...], 0.0)
        ev = _ln(_dot(mid, we2_r[...]) + be2_r[...], g_r[...], b_r[...])
        e_r[...] = jnp.concatenate([ev, jnp.zeros_like(ev)], axis=-1)

    return pl.pallas_call(
        body, grid=(n // r,),
        in_specs=[_rows(ef, r)] + [_rep(a) for a in (we1, be1, we2, be2, g, b)],
        out_specs=pl.BlockSpec((r, 2 * lat), lambda i: (i, 0)),
        out_shape=jax.ShapeDtypeStruct((n, 2 * lat), jnp.float32),
    )(ef, we1, be1, we2, be2, g, b)


def _edge_blk(gsum, e, w1c, w2, b2, g, b):
    n = e.shape[0]
    lat = w2.shape[1]
    r = _ROWS

    def body(gs_r, e_r, w1c_r, w2_r, b2_r, g_r, b_r, eo_r):
        ev = e_r[...][:, :lat]
        mid = jnp.maximum(gs_r[...] + _dot(ev, w1c_r[...]), 0.0)
        eo = ev + _ln(_dot(mid, w2_r[...]) + b2_r[...], g_r[...], b_r[...])
        eo_r[...] = jnp.concatenate([eo, jnp.zeros_like(eo)], axis=-1)

    return pl.pallas_call(
        body, grid=(n // r,),
        in_specs=[_rows(gsum, r), _rows(e, r)] +
                 [_rep(a) for a in (w1c, w2, b2, g, b)],
        out_specs=pl.BlockSpec((r, 2 * lat), lambda i: (i, 0)),
        out_shape=jax.ShapeDtypeStruct((n, 2 * lat), jnp.float32),
    )(gsum, e, w1c, w2, b2, g, b)


def _edge_blk_dec(gsum, e, w1c, w2, b2, g, b, wd1, bd1, wd2, bd2):
    n = e.shape[0]
    lat = w2.shape[1]
    dout = wd2.shape[1]
    r = _ROWS

    def body(gs_r, e_r, w1c_r, w2_r, b2_r, g_r, b_r, wd1_r, bd1_r, wd2_r, bd2_r,
             eo_r, de_r):
        ev = e_r[...][:, :lat]
        mid = jnp.maximum(gs_r[...] + _dot(ev, w1c_r[...]), 0.0)
        eo = ev + _ln(_dot(mid, w2_r[...]) + b2_r[...], g_r[...], b_r[...])
        eo_r[...] = jnp.concatenate([eo, jnp.zeros_like(eo)], axis=-1)
        d = jnp.maximum(_dot(eo, wd1_r[...]) + bd1_r[...], 0.0)
        de_r[...] = _dot(d, wd2_r[...]) + bd2_r[...]

    return pl.pallas_call(
        body, grid=(n // r,),
        in_specs=[_rows(gsum, r), _rows(e, r)] +
                 [_rep(a) for a in (w1c, w2, b2, g, b, wd1, bd1, wd2, bd2)],
        out_specs=[pl.BlockSpec((r, 2 * lat), lambda i: (i, 0)),
                   pl.BlockSpec((r, dout), lambda i: (i, 0))],
        out_shape=[jax.ShapeDtypeStruct((n, 2 * lat), jnp.float32),
                   jax.ShapeDtypeStruct((n, dout), jnp.float32)],
    )(gsum, e, w1c, w2, b2, g, b, wd1, bd1, wd2, bd2)


def _node_blk(h, parts, w1h, w1g, b1, w2, b2, g, b, nwab, nb1ab):
    n, lat = h.shape
    r = _ROWS

    def body(h_r, p_r, w1h_r, w1g_r, b1_r, w2_r, b2_r, g_r, b_r,
             wab_r, nb1ab_r, ho_r, hab_r):
        hv = h_r[...]
        agg = p_r[0, :, :lat] + p_r[1, :, :lat]
        mid = jnp.maximum(_dot(hv, w1h_r[...]) + _dot(agg, w1g_r[...]) + b1_r[...], 0.0)
        hn = hv + _ln(_dot(mid, w2_r[...]) + b2_r[...], g_r[...], b_r[...])
        ho_r[...] = hn
        hab_r[...] = _dot(hn, wab_r[...]) + nb1ab_r[...]

    return pl.pallas_call(
        body, grid=(n // r,),
        in_specs=[_rows(h, r), pl.BlockSpec((_NC, r, 2 * lat), lambda i: (0, i, 0))] +
                 [_rep(a) for a in (w1h, w1g, b1, w2, b2, g, b, nwab, nb1ab)],
        out_specs=[pl.BlockSpec((r, lat), lambda i: (i, 0)),
                   pl.BlockSpec((r, 2 * lat), lambda i: (i, 0))],
        out_shape=[jax.ShapeDtypeStruct((n, lat), jnp.float32),
                   jax.ShapeDtypeStruct((n, 2 * lat), jnp.float32)],
    )(h, parts, w1h, w1g, b1, w2, b2, g, b, nwab, nb1ab)


def _node_blk_dec(h, parts, w1h, w1g, b1, w2, b2, g, b, wd1, bd1, wd2, bd2):
    n, lat = h.shape
    dout = wd2.shape[1]
    r = _ROWS

    def body(h_r, p_r, w1h_r, w1g_r, b1_r, w2_r, b2_r, g_r, b_r,
             wd1_r, bd1_r, wd2_r, bd2_r, no_r):
        hv = h_r[...]
        agg = p_r[0, :, :lat] + p_r[1, :, :lat]
        mid = jnp.maximum(_dot(hv, w1h_r[...]) + _dot(agg, w1g_r[...]) + b1_r[...], 0.0)
        hn = hv + _ln(_dot(mid, w2_r[...]) + b2_r[...], g_r[...], b_r[...])
        d = jnp.maximum(_dot(hn, wd1_r[...]) + bd1_r[...], 0.0)
        no_r[...] = _dot(d, wd2_r[...]) + bd2_r[...]

    return pl.pallas_call(
        body, grid=(n // r,),
        in_specs=[_rows(h, r), pl.BlockSpec((_NC, r, 2 * lat), lambda i: (0, i, 0))] +
                 [_rep(a) for a in (w1h, w1g, b1, w2, b2, g, b, wd1, bd1, wd2, bd2)],
        out_specs=pl.BlockSpec((r, dout), lambda i: (i, 0)),
        out_shape=jax.ShapeDtypeStruct((n, dout), jnp.float32),
    )(h, parts, w1h, w1g, b1, w2, b2, g, b, wd1, bd1, wd2, bd2)


@functools.lru_cache(maxsize=None)
def _gather_fn(n_nodes, n_edges, lat):
    nw = _NC * _NS
    per_w = n_edges // nw
    nfull, tail = divmod(per_w, _CH)
    mesh = plsc.VectorSubcoreMesh(core_axis_name="c", subcore_axis_name="s")
    scratch = [pltpu.VMEM((per_w,), jnp.int32), pltpu.VMEM((per_w,), jnp.int32)]
    for _ in range(2):
        scratch += [pltpu.VMEM((_CH, 2 * lat), jnp.float32),
                    pltpu.VMEM((_CH, 2 * lat), jnp.float32),
                    pltpu.VMEM((_CH, lat), jnp.float32),
                    pltpu.SemaphoreType.DMA, pltpu.SemaphoreType.DMA]
    if tail:
        scratch += [pltpu.VMEM((tail, 2 * lat), jnp.float32),
                    pltpu.VMEM((tail, 2 * lat), jnp.float32),
                    pltpu.VMEM((tail, lat), jnp.float32)]
    scratch += [pltpu.SemaphoreType.DMA]

    @functools.partial(
        pl.kernel,
        out_type=jax.ShapeDtypeStruct((n_edges, lat), jnp.float32),
        mesh=mesh, scratch_types=scratch)
    def gather_k(hab, src, dst, out, *refs):
        ja, jb = refs[0], refs[1]
        sets = [refs[2:7], refs[7:12]]
        if tail:
            tba, tbb, tbo = refs[12:15]
        si = refs[-1]
        wid = lax.axis_index("s") * _NC + lax.axis_index("c")
        base0 = wid * per_w

        pltpu.async_copy(src.at[pl.ds(base0, per_w)], ja, si).wait()
        pltpu.async_copy(dst.at[pl.ds(base0, per_w)], jb, si).wait()

        def start(i, s):
            ba, bb, _, sa, sb = s
            pltpu.async_copy(hab.at[ja.at[pl.ds(i * _CH, _CH)]], ba, sa)
            pltpu.async_copy(hab.at[jb.at[pl.ds(i * _CH, _CH)]], bb, sb)

        def finish(i, s):
            ba, bb, bo, sa, sb = s
            pltpu.make_async_copy(hab.at[ja.at[pl.ds(i * _CH, _CH)]], ba, sa).wait()
            pltpu.make_async_copy(hab.at[jb.at[pl.ds(i * _CH, _CH)]], bb, sb).wait()

            @pl.loop(0, _CH)
            def _(rr):
                for cc in range(lat // 16):
                    bo[rr, pl.ds(cc * 16, 16)] = (
                        ba[rr, pl.ds(cc * 16, 16)]
                        + bb[rr, pl.ds(lat + cc * 16, 16)])

            pltpu.sync_copy(bo, out.at[pl.ds(base0 + i * _CH, _CH)])

        start(0, sets[0])
        if nfull > 1:
            start(1, sets[1])

        @pl.loop(0, nfull - nfull % 2, step=2)
        def _(i):
            finish(i, sets[0])

            @pl.when(i + 2 < nfull)
            def _():
                start(i + 2, sets[0])

            finish(i + 1, sets[1])

            @pl.when(i + 3 < nfull)
            def _():
                start(i + 3, sets[1])

        if nfull % 2:
            finish(nfull - 1, sets[0])

        if tail:
            jt = ja.at[pl.ds(nfull * _CH, tail)]
            kt = jb.at[pl.ds(nfull * _CH, tail)]
            pltpu.async_copy(hab.at[jt], tba, sets[0][3])
            pltpu.async_copy(hab.at[kt], tbb, sets[0][4])
            pltpu.make_async_copy(hab.at[jt], tba, sets[0][3]).wait()
            pltpu.make_async_copy(hab.at[kt], tbb, sets[0][4]).wait()

            @pl.loop(0, tail)
            def _(rr):
                for cc in range(lat // 16):
                    tbo[rr, pl.ds(cc * 16, 16)] = (
                        tba[rr, pl.ds(cc * 16, 16)]
                        + tbb[rr, pl.ds(lat + cc * 16, 16)])

            pltpu.sync_copy(tbo, out.at[pl.ds(base0 + nfull * _CH, tail)])

    return gather_k


@functools.lru_cache(maxsize=None)
def _segsum_fn(n_nodes, n_edges, lat):
    per_sc = n_edges // _NC
    per_w = per_sc // _NS
    nfull, tail = divmod(per_w, _CH)
    w = 2 * lat
    mesh = plsc.VectorSubcoreMesh(core_axis_name="c", subcore_axis_name="s")
    scratch = [pltpu.VMEM_SHARED((n_nodes, w), jnp.float32)]
    for _ in range(2):
        scratch += [pltpu.VMEM((_CH, w), jnp.float32),
                    pltpu.VMEM((_CH,), jnp.int32),
                    pltpu.SemaphoreType.DMA, pltpu.SemaphoreType.DMA]
    if tail:
        scratch += [pltpu.VMEM((tail, w), jnp.float32),
                    pltpu.VMEM((tail,), jnp.int32)]

    @functools.partial(
        pl.kernel,
        out_type=jax.ShapeDtypeStruct((_NC, n_nodes, w), jnp.float32),
        mesh=mesh, scratch_types=scratch)
    def segsum_k(e, dst, zeros, out, *refs):
        acc = refs[0]
        sets = [refs[1:5], refs[5:9]]
        if tail:
            tb, tj = refs[9], refs[10]
        cid = lax.axis_index("c")
        sid = lax.axis_index("s")

        @pl.when(sid == 0)
        def _():
            pltpu.sync_copy(zeros, acc)

        base0 = cid * per_sc + sid * per_w
        plsc.subcore_barrier()

        def start(i, s):
            bf, jx, se, si = s
            pltpu.async_copy(e.at[pl.ds(base0 + i * _CH, _CH)], bf, se)
            pltpu.async_copy(dst.at[pl.ds(base0 + i * _CH, _CH)], jx, si)

        def finish(i, s):
            bf, jx, se, si = s
            pltpu.make_async_copy(
                e.at[pl.ds(base0 + i * _CH, _CH)], bf, se).wait()
            pltpu.make_async_copy(
                dst.at[pl.ds(base0 + i * _CH, _CH)], jx, si).wait()
            pltpu.sync_copy(bf, acc.at[jx], add=True)

        start(0, sets[0])
        if nfull > 1:
            start(1, sets[1])

        @pl.loop(0, nfull - nfull % 2, step=2)
        def _(i):
            finish(i, sets[0])

            @pl.when(i + 2 < nfull)
            def _():
                start(i + 2, sets[0])

            finish(i + 1, sets[1])

            @pl.when(i + 3 < nfull)
            def _():
                start(i + 3, sets[1])

        if nfull % 2:
            finish(nfull - 1, sets[0])

        if tail:
            base_t = base0 + nfull * _CH
            pltpu.async_copy(e.at[pl.ds(base_t, tail)], tb, sets[0][2])
            pltpu.async_copy(dst.at[pl.ds(base_t, tail)], tj, sets[0][3])
            pltpu.make_async_copy(e.at[pl.ds(base_t, tail)], tb, sets[0][2]).wait()
            pltpu.make_async_copy(dst.at[pl.ds(base_t, tail)], tj, sets[0][3]).wait()
            pltpu.sync_copy(tb, acc.at[tj], add=True)

        plsc.subcore_barrier()

        @pl.when(sid == 0)
        def _():
            pltpu.sync_copy(acc, out.at[cid])

    return segsum_k


def kernel(x, edge_index, edge_features, params):
    src = edge_index[0]
    dst = edge_index[1]
    n_nodes = x.shape[0]
    n_edges = src.shape[0]

    def v(t):
        return t.reshape(1, -1)

    blocks = params["blocks"]
    nblk = len(blocks)

    en = params["enc_node"]
    (wn1, bn1), (wn2, bn2) = en["layers"]
    gn, betan = en["ln"]
    lat = wn2.shape[1]

    def packed_w1(i):
        w1 = blocks[i]["edge"]["layers"][0][0]
        b1 = blocks[i]["edge"]["layers"][0][1]
        wab = jnp.concatenate([w1[:lat], w1[lat:2 * lat]], axis=1)
        bab = jnp.concatenate([b1, jnp.zeros_like(b1)]).reshape(1, -1)
        return wab, bab

    wab0, bab0 = packed_w1(0)
    h, hab = _enc_node(x, wn1, v(bn1), wn2, v(bn2), v(gn), v(betan),
                       wab0, bab0)

    ee = params["enc_edge"]
    (we1, be1), (we2, be2) = ee["layers"]
    ge, betae = ee["ln"]
    e = _enc_edge(edge_features, we1, v(be1), we2, v(be2), v(ge), v(betae))

    zeros = jnp.zeros((n_nodes, 2 * lat), jnp.float32)
    (wdn1, bdn1), (wdn2, bdn2) = params["dec_node"]["layers"]
    (wde1, bde1), (wde2, bde2) = params["dec_edge"]["layers"]

    gather = _gather_fn(n_nodes, n_edges, lat)
    segsum = _segsum_fn(n_nodes, n_edges, lat)

    node_out = edge_out = None
    for i in range(nblk):
        blk = blocks[i]
        (ew1, _), (ew2, eb2) = blk["edge"]["layers"]
        eg, ebeta = blk["edge"]["ln"]
        w1c = ew1[2 * lat:]

        gsum = gather(hab, src, dst)
        if i < nblk - 1:
            e = _edge_blk(gsum, e, w1c, ew2, v(eb2), v(eg), v(ebeta))
        else:
            e, edge_out = _edge_blk_dec(gsum, e, w1c, ew2, v(eb2), v(eg),
                                        v(ebeta), wde1, v(bde1), wde2, v(bde2))

        parts = segsum(e, dst, zeros)

        (nw1, nb1), (nw2, nb2) = blk["node"]["layers"]
        ng, nbeta = blk["node"]["ln"]
        w1h, w1g = nw1[:lat], nw1[lat:]
        if i < nblk - 1:
            nwab, nbab = packed_w1(i + 1)
            h, hab = _node_blk(h, parts, w1h, w1g, v(nb1), nw2, v(nb2),
                               v(ng), v(nbeta), nwab, nbab)
        else:
            node_out = _node_blk_dec(h, parts, w1h, w1g, v(nb1), nw2, v(nb2),
                                     v(ng), v(nbeta),
                                     wdn1, v(bdn1), wdn2, v(bdn2))

    return (node_out, edge_out)


# speedup vs baseline: 1.0808x; 1.0808x over previous
import functools

import jax
import jax.numpy as jnp
from jax import lax
from jax.experimental import pallas as pl
from jax.experimental.pallas import tpu as pltpu
from jax.experimental.pallas import tpu_sc as plsc

_EPS = 1e-5
_NC, _NS = 2, 16
_CH = 128
_ROWS = 2000


def _ln(y, g, b):
    mu = jnp.mean(y, axis=-1, keepdims=True)
    var = jnp.mean((y - mu) ** 2, axis=-1, keepdims=True)
    return (y - mu) * lax.rsqrt(var + _EPS) * g + b


def _dot(a, b):
    return jnp.dot(a, b, preferred_element_type=jnp.float32)


def _rep(a):
    return pl.BlockSpec(a.shape, lambda i: (0,) * a.ndim)


def _rows(a, r):
    return pl.BlockSpec((r, a.shape[1]), lambda i: (i, 0))


def _enc_node(x, we1, be1, we2, be2, g, b, w1ab, b1ab):
    n = x.shape[0]
    lat = we2.shape[1]
    r = _ROWS

    def body(x_r, we1_r, be1_r, we2_r, be2_r, g_r, b_r, wab_r, b1ab_r,
             h_r, hab_r):
        xx = x_r[...]
        mid = jnp.maximum(_dot(xx, we1_r[...]) + be1_r[...], 0.0)
        h = _ln(_dot(mid, we2_r[...]) + be2_r[...], g_r[...], b_r[...])
        h_r[...] = h
        hab_r[...] = _dot(h, wab_r[...]) + b1ab_r[...]

    return pl.pallas_call(
        body, grid=(n // r,),
        in_specs=[_rows(x, r)] + [_rep(a) for a in (we1, be1, we2, be2, g, b,
                                                    w1ab, b1ab)],
        out_specs=[pl.BlockSpec((r, lat), lambda i: (i, 0)),
                   pl.BlockSpec((r, 2 * lat), lambda i: (i, 0))],
        out_shape=[jax.ShapeDtypeStruct((n, lat), jnp.float32),
                   jax.ShapeDtypeStruct((n, 2 * lat), jnp.float32)],
    )(x, we1, be1, we2, be2, g, b, w1ab, b1ab)


def _enc_edge(ef, we1, be1, we2, be2, g, b, off, nrows):
    lat = we2.shape[1]
    r = _ROWS
    ot = off // r

    def body(ef_r, we1_r, be1_r, we2_r, be2_r, g_r, b_r, e_r):
        mid = jnp.maximum(_dot(ef_r[...], we1_r[...]) + be1_r[...], 0.0)
        ev = _ln(_dot(mid, we2_r[...]) + be2_r[...], g_r[...], b_r[...])
        e_r[...] = jnp.concatenate([ev, jnp.zeros_like(ev)], axis=-1)

    return pl.pallas_call(
        body, grid=(nrows // r,),
        in_specs=[pl.BlockSpec((r, ef.shape[1]), lambda i: (i + ot, 0))] +
                 [_rep(a) for a in (we1, be1, we2, be2, g, b)],
        out_specs=pl.BlockSpec((r, 2 * lat), lambda i: (i, 0)),
        out_shape=jax.ShapeDtypeStruct((nrows, 2 * lat), jnp.float32),
    )(ef, we1, be1, we2, be2, g, b)


def _edge_blk(gsum, e, w1c, w2, b2, g, b):
    n = e.shape[0]
    lat = w2.shape[1]
    r = _ROWS

    def body(gs_r, e_r, w1c_r, w2_r, b2_r, g_r, b_r, eo_r):
        ev = e_r[...][:, :lat]
        mid = jnp.maximum(gs_r[...] + _dot(ev, w1c_r[...]), 0.0)
        eo = ev + _ln(_dot(mid, w2_r[...]) + b2_r[...], g_r[...], b_r[...])
        eo_r[...] = jnp.concatenate([eo, jnp.zeros_like(eo)], axis=-1)

    return pl.pallas_call(
        body, grid=(n // r,),
        in_specs=[_rows(gsum, r), _rows(e, r)] +
                 [_rep(a) for a in (w1c, w2, b2, g, b)],
        out_specs=pl.BlockSpec((r, 2 * lat), lambda i: (i, 0)),
        out_shape=jax.ShapeDtypeStruct((n, 2 * lat), jnp.float32),
    )(gsum, e, w1c, w2, b2, g, b)


def _edge_blk_dec(gsum, e, w1c, w2, b2, g, b, wd1, bd1, wd2, bd2):
    n = e.shape[0]
    lat = w2.shape[1]
    dout = wd2.shape[1]
    r = _ROWS

    def body(gs_r, e_r, w1c_r, w2_r, b2_r, g_r, b_r, wd1_r, bd1_r, wd2_r, bd2_r,
             eo_r, de_r):
        ev = e_r[...][:, :lat]
        mid = jnp.maximum(gs_r[...] + _dot(ev, w1c_r[...]), 0.0)
        eo = ev + _ln(_dot(mid, w2_r[...]) + b2_r[...], g_r[...], b_r[...])
        eo_r[...] = jnp.concatenate([eo, jnp.zeros_like(eo)], axis=-1)
        d = jnp.maximum(_dot(eo, wd1_r[...]) + bd1_r[...], 0.0)
        de_r[...] = _dot(d, wd2_r[...]) + bd2_r[...]

    return pl.pallas_call(
        body, grid=(n // r,),
        in_specs=[_rows(gsum, r), _rows(e, r)] +
                 [_rep(a) for a in (w1c, w2, b2, g, b, wd1, bd1, wd2, bd2)],
        out_specs=[pl.BlockSpec((r, 2 * lat), lambda i: (i, 0)),
                   pl.BlockSpec((r, dout), lambda i: (i, 0))],
        out_shape=[jax.ShapeDtypeStruct((n, 2 * lat), jnp.float32),
                   jax.ShapeDtypeStruct((n, dout), jnp.float32)],
    )(gsum, e, w1c, w2, b2, g, b, wd1, bd1, wd2, bd2)


def _node_blk(h, pa, pb, w1h, w1g, b1, w2, b2, g, b, nwab, nb1ab):
    n, lat = h.shape
    r = _ROWS

    def body(h_r, pa_r, pb_r, w1h_r, w1g_r, b1_r, w2_r, b2_r, g_r, b_r,
             wab_r, nb1ab_r, ho_r, hab_r):
        hv = h_r[...]
        agg = (pa_r[0, :, :lat] + pa_r[1, :, :lat]
               + pb_r[0, :, :lat] + pb_r[1, :, :lat])
        mid = jnp.maximum(_dot(hv, w1h_r[...]) + _dot(agg, w1g_r[...]) + b1_r[...], 0.0)
        hn = hv + _ln(_dot(mid, w2_r[...]) + b2_r[...], g_r[...], b_r[...])
        ho_r[...] = hn
        hab_r[...] = _dot(hn, wab_r[...]) + nb1ab_r[...]

    p_spec = pl.BlockSpec((_NC, r, 2 * lat), lambda i: (0, i, 0))
    return pl.pallas_call(
        body, grid=(n // r,),
        in_specs=[_rows(h, r), p_spec, p_spec] +
                 [_rep(a) for a in (w1h, w1g, b1, w2, b2, g, b, nwab, nb1ab)],
        out_specs=[pl.BlockSpec((r, lat), lambda i: (i, 0)),
                   pl.BlockSpec((r, 2 * lat), lambda i: (i, 0))],
        out_shape=[jax.ShapeDtypeStruct((n, lat), jnp.float32),
                   jax.ShapeDtypeStruct((n, 2 * lat), jnp.float32)],
    )(h, pa, pb, w1h, w1g, b1, w2, b2, g, b, nwab, nb1ab)


def _node_blk_dec(h, pa, pb, w1h, w1g, b1, w2, b2, g, b, wd1, bd1, wd2, bd2):
    n, lat = h.shape
    dout = wd2.shape[1]
    r = _ROWS

    def body(h_r, pa_r, pb_r, w1h_r, w1g_r, b1_r, w2_r, b2_r, g_r, b_r,
             wd1_r, bd1_r, wd2_r, bd2_r, no_r):
        hv = h_r[...]
        agg = (pa_r[0, :, :lat] + pa_r[1, :, :lat]
               + pb_r[0, :, :lat] + pb_r[1, :, :lat])
        mid = jnp.maximum(_dot(hv, w1h_r[...]) + _dot(agg, w1g_r[...]) + b1_r[...], 0.0)
        hn = hv + _ln(_dot(mid, w2_r[...]) + b2_r[...], g_r[...], b_r[...])
        d = jnp.maximum(_dot(hn, wd1_r[...]) + bd1_r[...], 0.0)
        no_r[...] = _dot(d, wd2_r[...]) + bd2_r[...]

    p_spec = pl.BlockSpec((_NC, r, 2 * lat), lambda i: (0, i, 0))
    return pl.pallas_call(
        body, grid=(n // r,),
        in_specs=[_rows(h, r), p_spec, p_spec] +
                 [_rep(a) for a in (w1h, w1g, b1, w2, b2, g, b, wd1, bd1, wd2, bd2)],
        out_specs=pl.BlockSpec((r, dout), lambda i: (i, 0)),
        out_shape=jax.ShapeDtypeStruct((n, dout), jnp.float32),
    )(h, pa, pb, w1h, w1g, b1, w2, b2, g, b, wd1, bd1, wd2, bd2)


@functools.lru_cache(maxsize=None)
def _gather_fn(n_nodes, n_edges, lat):
    nw = _NC * _NS
    per_w = n_edges // nw
    nfull, tail = divmod(per_w, _CH)
    mesh = plsc.VectorSubcoreMesh(core_axis_name="c", subcore_axis_name="s")
    scratch = [pltpu.VMEM((per_w,), jnp.int32), pltpu.VMEM((per_w,), jnp.int32)]
    for _ in range(2):
        scratch += [pltpu.VMEM((_CH, 2 * lat), jnp.float32),
                    pltpu.VMEM((_CH, 2 * lat), jnp.float32),
                    pltpu.VMEM((_CH, lat), jnp.float32),
                    pltpu.SemaphoreType.DMA, pltpu.SemaphoreType.DMA]
    if tail:
        scratch += [pltpu.VMEM((tail, 2 * lat), jnp.float32),
                    pltpu.VMEM((tail, 2 * lat), jnp.float32),
                    pltpu.VMEM((tail, lat), jnp.float32)]
    scratch += [pltpu.SemaphoreType.DMA]

    @functools.partial(
        pl.kernel,
        out_type=jax.ShapeDtypeStruct((n_edges, lat), jnp.float32),
        mesh=mesh, scratch_types=scratch)
    def gather_k(hab, src, dst, out, *refs):
        ja, jb = refs[0], refs[1]
        sets = [refs[2:7], refs[7:12]]
        if tail:
            tba, tbb, tbo = refs[12:15]
        si = refs[-1]
        wid = lax.axis_index("s") * _NC + lax.axis_index("c")
        base0 = wid * per_w

        pltpu.async_copy(src.at[pl.ds(base0, per_w)], ja, si).wait()
        pltpu.async_copy(dst.at[pl.ds(base0, per_w)], jb, si).wait()

        def start(i, s):
            ba, bb, _, sa, sb = s
            pltpu.async_copy(hab.at[ja.at[pl.ds(i * _CH, _CH)]], ba, sa)
            pltpu.async_copy(hab.at[jb.at[pl.ds(i * _CH, _CH)]], bb, sb)

        def finish(i, s):
            ba, bb, bo, sa, sb = s
            pltpu.make_async_copy(hab.at[ja.at[pl.ds(i * _CH, _CH)]], ba, sa).wait()
            pltpu.make_async_copy(hab.at[jb.at[pl.ds(i * _CH, _CH)]], bb, sb).wait()

            @pl.loop(0, _CH)
            def _(rr):
                for cc in range(lat // 16):
                    bo[rr, pl.ds(cc * 16, 16)] = (
                        ba[rr, pl.ds(cc * 16, 16)]
                        + bb[rr, pl.ds(lat + cc * 16, 16)])

            pltpu.sync_copy(bo, out.at[pl.ds(base0 + i * _CH, _CH)])

        start(0, sets[0])
        if nfull > 1:
            start(1, sets[1])

        @pl.loop(0, nfull - nfull % 2, step=2)
        def _(i):
            finish(i, sets[0])

            @pl.when(i + 2 < nfull)
            def _():
                start(i + 2, sets[0])

            finish(i + 1, sets[1])

            @pl.when(i + 3 < nfull)
            def _():
                start(i + 3, sets[1])

        if nfull % 2:
            finish(nfull - 1, sets[0])

        if tail:
            jt = ja.at[pl.ds(nfull * _CH, tail)]
            kt = jb.at[pl.ds(nfull * _CH, tail)]
            pltpu.async_copy(hab.at[jt], tba, sets[0][3])
            pltpu.async_copy(hab.at[kt], tbb, sets[0][4])
            pltpu.make_async_copy(hab.at[jt], tba, sets[0][3]).wait()
            pltpu.make_async_copy(hab.at[kt], tbb, sets[0][4]).wait()

            @pl.loop(0, tail)
            def _(rr):
                for cc in range(lat // 16):
                    tbo[rr, pl.ds(cc * 16, 16)] = (
                        tba[rr, pl.ds(cc * 16, 16)]
                        + tbb[rr, pl.ds(lat + cc * 16, 16)])

            pltpu.sync_copy(tbo, out.at[pl.ds(base0 + nfull * _CH, tail)])

    return gather_k


@functools.lru_cache(maxsize=None)
def _segsum_fn(n_nodes, n_edges, lat):
    per_sc = n_edges // _NC
    per_w = per_sc // _NS
    nfull, tail = divmod(per_w, _CH)
    w = 2 * lat
    mesh = plsc.VectorSubcoreMesh(core_axis_name="c", subcore_axis_name="s")
    scratch = [pltpu.VMEM_SHARED((n_nodes, w), jnp.float32)]
    for _ in range(2):
        scratch += [pltpu.VMEM((_CH, w), jnp.float32),
                    pltpu.VMEM((_CH,), jnp.int32),
                    pltpu.SemaphoreType.DMA, pltpu.SemaphoreType.DMA]
    if tail:
        scratch += [pltpu.VMEM((tail, w), jnp.float32),
                    pltpu.VMEM((tail,), jnp.int32)]

    @functools.partial(
        pl.kernel,
        out_type=jax.ShapeDtypeStruct((_NC, n_nodes, w), jnp.float32),
        mesh=mesh, scratch_types=scratch)
    def segsum_k(e, dst, zeros, out, *refs):
        acc = refs[0]
        sets = [refs[1:5], refs[5:9]]
        if tail:
            tb, tj = refs[9], refs[10]
        cid = lax.axis_index("c")
        sid = lax.axis_index("s")

        @pl.when(sid == 0)
        def _():
            pltpu.sync_copy(zeros, acc)

        base0 = cid * per_sc + sid * per_w
        plsc.subcore_barrier()

        def start(i, s):
            bf, jx, se, si = s
            pltpu.async_copy(e.at[pl.ds(base0 + i * _CH, _CH)], bf, se)
            pltpu.async_copy(dst.at[pl.ds(base0 + i * _CH, _CH)], jx, si)

        def finish(i, s):
            bf, jx, se, si = s
            pltpu.make_async_copy(
                e.at[pl.ds(base0 + i * _CH, _CH)], bf, se).wait()
            pltpu.make_async_copy(
                dst.at[pl.ds(base0 + i * _CH, _CH)], jx, si).wait()
            pltpu.sync_copy(bf, acc.at[jx], add=True)

        start(0, sets[0])
        if nfull > 1:
            start(1, sets[1])

        @pl.loop(0, nfull - nfull % 2, step=2)
        def _(i):
            finish(i, sets[0])

            @pl.when(i + 2 < nfull)
            def _():
                start(i + 2, sets[0])

            finish(i + 1, sets[1])

            @pl.when(i + 3 < nfull)
            def _():
                start(i + 3, sets[1])

        if nfull % 2:
            finish(nfull - 1, sets[0])

        if tail:
            base_t = base0 + nfull * _CH
            pltpu.async_copy(e.at[pl.ds(base_t, tail)], tb, sets[0][2])
            pltpu.async_copy(dst.at[pl.ds(base_t, tail)], tj, sets[0][3])
            pltpu.make_async_copy(e.at[pl.ds(base_t, tail)], tb, sets[0][2]).wait()
            pltpu.make_async_copy(dst.at[pl.ds(base_t, tail)], tj, sets[0][3]).wait()
            pltpu.sync_copy(tb, acc.at[tj], add=True)

        plsc.subcore_barrier()

        @pl.when(sid == 0)
        def _():
            pltpu.sync_copy(acc, out.at[cid])

    return segsum_k


def kernel(x, edge_index, edge_features, params):
    src = edge_index[0]
    dst = edge_index[1]
    n_nodes = x.shape[0]
    n_edges = src.shape[0]

    def v(t):
        return t.reshape(1, -1)

    blocks = params["blocks"]
    nblk = len(blocks)

    en = params["enc_node"]
    (wn1, bn1), (wn2, bn2) = en["layers"]
    gn, betan = en["ln"]
    lat = wn2.shape[1]

    def packed_w1(i):
        w1 = blocks[i]["edge"]["layers"][0][0]
        b1 = blocks[i]["edge"]["layers"][0][1]
        wab = jnp.concatenate([w1[:lat], w1[lat:2 * lat]], axis=1)
        bab = jnp.concatenate([b1, jnp.zeros_like(b1)]).reshape(1, -1)
        return wab, bab

    wab0, bab0 = packed_w1(0)
    h, hab = _enc_node(x, wn1, v(bn1), wn2, v(bn2), v(gn), v(betan),
                       wab0, bab0)

    ee = params["enc_edge"]
    (we1, be1), (we2, be2) = ee["layers"]
    ge, betae = ee["ln"]
    half = n_edges // 2
    eh = [_enc_edge(edge_features, we1, v(be1), we2, v(be2), v(ge), v(betae),
                    o, half) for o in (0, half)]

    src_h = [lax.slice_in_dim(src, 0, half), lax.slice_in_dim(src, half, n_edges)]
    dst_h = [lax.slice_in_dim(dst, 0, half), lax.slice_in_dim(dst, half, n_edges)]

    zeros = jnp.zeros((n_nodes, 2 * lat), jnp.float32)
    (wdn1, bdn1), (wdn2, bdn2) = params["dec_node"]["layers"]
    (wde1, bde1), (wde2, bde2) = params["dec_edge"]["layers"]

    gather = _gather_fn(n_nodes, half, lat)
    segsum = _segsum_fn(n_nodes, half, lat)

    node_out = edge_out = None
    for i in range(nblk):
        blk = blocks[i]
        (ew1, _), (ew2, eb2) = blk["edge"]["layers"]
        eg, ebeta = blk["edge"]["ln"]
        w1c = ew1[2 * lat:]

        parts = [None, None]
        dec_h = [None, None]
        for s in (0, 1):
            gsum = gather(hab, src_h[s], dst_h[s])
            if i < nblk - 1:
                eh[s] = _edge_blk(gsum, eh[s], w1c, ew2, v(eb2), v(eg),
                                  v(ebeta))
            else:
                eh[s], dec_h[s] = _edge_blk_dec(
                    gsum, eh[s], w1c, ew2, v(eb2), v(eg), v(ebeta),
                    wde1, v(bde1), wde2, v(bde2))
            parts[s] = segsum(eh[s], dst_h[s], zeros)

        (nw1, nb1), (nw2, nb2) = blk["node"]["layers"]
        ng, nbeta = blk["node"]["ln"]
        w1h, w1g = nw1[:lat], nw1[lat:]
        if i < nblk - 1:
            nwab, nbab = packed_w1(i + 1)
            h, hab = _node_blk(h, parts[0], parts[1], w1h, w1g, v(nb1), nw2,
                               v(nb2), v(ng), v(nbeta), nwab, nbab)
        else:
            node_out = _node_blk_dec(h, parts[0], parts[1], w1h, w1g, v(nb1),
                                     nw2, v(nb2), v(ng), v(nbeta),
                                     wdn1, v(bdn1), wdn2, v(bdn2))
            edge_out = jnp.concatenate(dec_h, axis=0)

    return (node_out, edge_out)


# speedup vs baseline: 1.0813x; 1.0004x over previous
import functools

import jax
import jax.numpy as jnp
from jax import lax
from jax.experimental import pallas as pl
from jax.experimental.pallas import tpu as pltpu
from jax.experimental.pallas import tpu_sc as plsc

_EPS = 1e-5
_NC, _NS = 2, 16
_CH = 128
_ROWS = 2000


def _ln(y, g, b):
    mu = jnp.mean(y, axis=-1, keepdims=True)
    var = jnp.mean((y - mu) ** 2, axis=-1, keepdims=True)
    return (y - mu) * lax.rsqrt(var + _EPS) * g + b


def _dot(a, b):
    return jnp.dot(a, b, preferred_element_type=jnp.float32)


def _rep(a):
    return pl.BlockSpec(a.shape, lambda i: (0,) * a.ndim)


def _rows(a, r):
    return pl.BlockSpec((r, a.shape[1]), lambda i: (i, 0))


def _enc_node(x, we1, be1, we2, be2, g, b, w1ab, b1ab):
    n = x.shape[0]
    lat = we2.shape[1]
    r = _ROWS

    def body(x_r, we1_r, be1_r, we2_r, be2_r, g_r, b_r, wab_r, b1ab_r,
             h_r, hab_r):
        xx = x_r[...]
        mid = jnp.maximum(_dot(xx, we1_r[...]) + be1_r[...], 0.0)
        h = _ln(_dot(mid, we2_r[...]) + be2_r[...], g_r[...], b_r[...])
        h_r[...] = h
        hab_r[...] = _dot(h, wab_r[...]) + b1ab_r[...]

    return pl.pallas_call(
        body, grid=(n // r,),
        in_specs=[_rows(x, r)] + [_rep(a) for a in (we1, be1, we2, be2, g, b,
                                                    w1ab, b1ab)],
        out_specs=[pl.BlockSpec((r, lat), lambda i: (i, 0)),
                   pl.BlockSpec((r, 2 * lat), lambda i: (i, 0))],
        out_shape=[jax.ShapeDtypeStruct((n, lat), jnp.float32),
                   jax.ShapeDtypeStruct((n, 2 * lat), jnp.float32)],
    )(x, we1, be1, we2, be2, g, b, w1ab, b1ab)


def _enc_edge(ef, we1, be1, we2, be2, g, b, off, nrows):
    lat = we2.shape[1]
    r = _ROWS
    ot = off // r

    def body(ef_r, we1_r, be1_r, we2_r, be2_r, g_r, b_r, e_r):
        mid = jnp.maximum(_dot(ef_r[...], we1_r[...]) + be1_r[...], 0.0)
        ev = _ln(_dot(mid, we2_r[...]) + be2_r[...], g_r[...], b_r[...])
        e_r[...] = jnp.concatenate([ev, jnp.zeros_like(ev)], axis=-1)

    return pl.pallas_call(
        body, grid=(nrows // r,),
        in_specs=[pl.BlockSpec((r, ef.shape[1]), lambda i: (i + ot, 0))] +
                 [_rep(a) for a in (we1, be1, we2, be2, g, b)],
        out_specs=pl.BlockSpec((r, 2 * lat), lambda i: (i, 0)),
        out_shape=jax.ShapeDtypeStruct((nrows, 2 * lat), jnp.float32),
    )(ef, we1, be1, we2, be2, g, b)


def _edge_blk(gsum, e, w1c, w2, b2, g, b):
    n = e.shape[0]
    lat = w2.shape[1]
    r = _ROWS

    def body(gs_r, e_r, w1c_r, w2_r, b2_r, g_r, b_r, eo_r):
        ev = e_r[...][:, :lat]
        mid = jnp.maximum(gs_r[...] + _dot(ev, w1c_r[...]), 0.0)
        eo = ev + _ln(_dot(mid, w2_r[...]) + b2_r[...], g_r[...], b_r[...])
        eo_r[...] = jnp.concatenate([eo, jnp.zeros_like(eo)], axis=-1)

    return pl.pallas_call(
        body, grid=(n // r,),
        in_specs=[_rows(gsum, r), _rows(e, r)] +
                 [_rep(a) for a in (w1c, w2, b2, g, b)],
        out_specs=pl.BlockSpec((r, 2 * lat), lambda i: (i, 0)),
        out_shape=jax.ShapeDtypeStruct((n, 2 * lat), jnp.float32),
    )(gsum, e, w1c, w2, b2, g, b)


def _edge_blk_dec(gsum, e, w1c, w2, b2, g, b, wd1, bd1, wd2, bd2):
    n = e.shape[0]
    lat = w2.shape[1]
    dout = wd2.shape[1]
    r = _ROWS

    def body(gs_r, e_r, w1c_r, w2_r, b2_r, g_r, b_r, wd1_r, bd1_r, wd2_r, bd2_r,
             eo_r, de_r):
        ev = e_r[...][:, :lat]
        mid = jnp.maximum(gs_r[...] + _dot(ev, w1c_r[...]), 0.0)
        eo = ev + _ln(_dot(mid, w2_r[...]) + b2_r[...], g_r[...], b_r[...])
        eo_r[...] = jnp.concatenate([eo, jnp.zeros_like(eo)], axis=-1)
        d = jnp.maximum(_dot(eo, wd1_r[...]) + bd1_r[...], 0.0)
        de_r[...] = _dot(d, wd2_r[...]) + bd2_r[...]

    return pl.pallas_call(
        body, grid=(n // r,),
        in_specs=[_rows(gsum, r), _rows(e, r)] +
                 [_rep(a) for a in (w1c, w2, b2, g, b, wd1, bd1, wd2, bd2)],
        out_specs=[pl.BlockSpec((r, 2 * lat), lambda i: (i, 0)),
                   pl.BlockSpec((r, dout), lambda i: (i, 0))],
        out_shape=[jax.ShapeDtypeStruct((n, 2 * lat), jnp.float32),
                   jax.ShapeDtypeStruct((n, dout), jnp.float32)],
    )(gsum, e, w1c, w2, b2, g, b, wd1, bd1, wd2, bd2)


def _node_blk(h, pa, pb, w1h, w1g, b1, w2, b2, g, b, nwab, nb1ab):
    n, lat = h.shape
    r = _ROWS

    def body(h_r, pa_r, pb_r, w1h_r, w1g_r, b1_r, w2_r, b2_r, g_r, b_r,
             wab_r, nb1ab_r, ho_r, hab_r):
        hv = h_r[...]
        agg = (pa_r[0, :, :lat] + pa_r[1, :, :lat]
               + pb_r[0, :, :lat] + pb_r[1, :, :lat])
        mid = jnp.maximum(_dot(hv, w1h_r[...]) + _dot(agg, w1g_r[...]) + b1_r[...], 0.0)
        hn = hv + _ln(_dot(mid, w2_r[...]) + b2_r[...], g_r[...], b_r[...])
        ho_r[...] = hn
        hab_r[...] = _dot(hn, wab_r[...]) + nb1ab_r[...]

    p_spec = pl.BlockSpec((_NC, r, 2 * lat), lambda i: (0, i, 0))
    return pl.pallas_call(
        body, grid=(n // r,),
        in_specs=[_rows(h, r), p_spec, p_spec] +
                 [_rep(a) for a in (w1h, w1g, b1, w2, b2, g, b, nwab, nb1ab)],
        out_specs=[pl.BlockSpec((r, lat), lambda i: (i, 0)),
                   pl.BlockSpec((r, 2 * lat), lambda i: (i, 0))],
        out_shape=[jax.ShapeDtypeStruct((n, lat), jnp.float32),
                   jax.ShapeDtypeStruct((n, 2 * lat), jnp.float32)],
    )(h, pa, pb, w1h, w1g, b1, w2, b2, g, b, nwab, nb1ab)


def _node_blk_dec(h, pa, pb, w1h, w1g, b1, w2, b2, g, b, wd1, bd1, wd2, bd2):
    n, lat = h.shape
    dout = wd2.shape[1]
    r = _ROWS

    def body(h_r, pa_r, pb_r, w1h_r, w1g_r, b1_r, w2_r, b2_r, g_r, b_r,
             wd1_r, bd1_r, wd2_r, bd2_r, no_r):
        hv = h_r[...]
        agg = (pa_r[0, :, :lat] + pa_r[1, :, :lat]
               + pb_r[0, :, :lat] + pb_r[1, :, :lat])
        mid = jnp.maximum(_dot(hv, w1h_r[...]) + _dot(agg, w1g_r[...]) + b1_r[...], 0.0)
        hn = hv + _ln(_dot(mid, w2_r[...]) + b2_r[...], g_r[...], b_r[...])
        d = jnp.maximum(_dot(hn, wd1_r[...]) + bd1_r[...], 0.0)
        no_r[...] = _dot(d, wd2_r[...]) + bd2_r[...]

    p_spec = pl.BlockSpec((_NC, r, 2 * lat), lambda i: (0, i, 0))
    return pl.pallas_call(
        body, grid=(n // r,),
        in_specs=[_rows(h, r), p_spec, p_spec] +
                 [_rep(a) for a in (w1h, w1g, b1, w2, b2, g, b, wd1, bd1, wd2, bd2)],
        out_specs=pl.BlockSpec((r, dout), lambda i: (i, 0)),
        out_shape=jax.ShapeDtypeStruct((n, dout), jnp.float32),
    )(h, pa, pb, w1h, w1g, b1, w2, b2, g, b, wd1, bd1, wd2, bd2)


@functools.lru_cache(maxsize=None)
def _gather_fn(n_nodes, n_edges, lat):
    nw = _NC * _NS
    per_w = n_edges // nw
    nfull, tail = divmod(per_w, _CH)
    mesh = plsc.VectorSubcoreMesh(core_axis_name="c", subcore_axis_name="s")
    scratch = [pltpu.VMEM((per_w,), jnp.int32), pltpu.VMEM((per_w,), jnp.int32)]
    for _ in range(2):
        scratch += [pltpu.VMEM((_CH, 2 * lat), jnp.float32),
                    pltpu.VMEM((_CH, 2 * lat), jnp.float32),
                    pltpu.VMEM((_CH, lat), jnp.float32),
                    pltpu.SemaphoreType.DMA, pltpu.SemaphoreType.DMA,
                    pltpu.SemaphoreType.DMA]
    if tail:
        scratch += [pltpu.VMEM((tail, 2 * lat), jnp.float32),
                    pltpu.VMEM((tail, 2 * lat), jnp.float32),
                    pltpu.VMEM((tail, lat), jnp.float32)]
    scratch += [pltpu.SemaphoreType.DMA]

    @functools.partial(
        pl.kernel,
        out_type=jax.ShapeDtypeStruct((n_edges, lat), jnp.float32),
        mesh=mesh, scratch_types=scratch)
    def gather_k(hab, src, dst, out, *refs):
        ja, jb = refs[0], refs[1]
        sets = [refs[2:8], refs[8:14]]
        if tail:
            tba, tbb, tbo = refs[14:17]
        si = refs[-1]
        wid = lax.axis_index("s") * _NC + lax.axis_index("c")
        base0 = wid * per_w

        pltpu.async_copy(src.at[pl.ds(base0, per_w)], ja, si).wait()
        pltpu.async_copy(dst.at[pl.ds(base0, per_w)], jb, si).wait()

        def start(i, s):
            ba, bb, _, sa, sb, _ = s
            pltpu.async_copy(hab.at[ja.at[pl.ds(i * _CH, _CH)]], ba, sa)
            pltpu.async_copy(hab.at[jb.at[pl.ds(i * _CH, _CH)]], bb, sb)

        def finish(i, s):
            ba, bb, bo, sa, sb, sw = s
            pltpu.make_async_copy(hab.at[ja.at[pl.ds(i * _CH, _CH)]], ba, sa).wait()
            pltpu.make_async_copy(hab.at[jb.at[pl.ds(i * _CH, _CH)]], bb, sb).wait()

            @pl.when(i > 1)
            def _():
                pltpu.make_async_copy(
                    bo, out.at[pl.ds(base0 + i * _CH, _CH)], sw).wait()

            @pl.loop(0, _CH)
            def _(rr):
                for cc in range(lat // 16):
                    bo[rr, pl.ds(cc * 16, 16)] = (
                        ba[rr, pl.ds(cc * 16, 16)]
                        + bb[rr, pl.ds(lat + cc * 16, 16)])

            pltpu.async_copy(bo, out.at[pl.ds(base0 + i * _CH, _CH)], sw)

        start(0, sets[0])
        if nfull > 1:
            start(1, sets[1])

        @pl.loop(0, nfull - nfull % 2, step=2)
        def _(i):
            finish(i, sets[0])

            @pl.when(i + 2 < nfull)
            def _():
                start(i + 2, sets[0])

            finish(i + 1, sets[1])

            @pl.when(i + 3 < nfull)
            def _():
                start(i + 3, sets[1])

        if nfull % 2:
            finish(nfull - 1, sets[0])

        if tail:
            jt = ja.at[pl.ds(nfull * _CH, tail)]
            kt = jb.at[pl.ds(nfull * _CH, tail)]
            pltpu.async_copy(hab.at[jt], tba, sets[0][3])
            pltpu.async_copy(hab.at[kt], tbb, sets[0][4])
            pltpu.make_async_copy(hab.at[jt], tba, sets[0][3]).wait()
            pltpu.make_async_copy(hab.at[kt], tbb, sets[0][4]).wait()

            @pl.loop(0, tail)
            def _(rr):
                for cc in range(lat // 16):
                    tbo[rr, pl.ds(cc * 16, 16)] = (
                        tba[rr, pl.ds(cc * 16, 16)]
                        + tbb[rr, pl.ds(lat + cc * 16, 16)])

            pltpu.sync_copy(tbo, out.at[pl.ds(base0 + nfull * _CH, tail)])

        if nfull > 0:
            pltpu.make_async_copy(
                sets[0][2], out.at[pl.ds(base0, _CH)], sets[0][5]).wait()
        if nfull > 1:
            pltpu.make_async_copy(
                sets[1][2], out.at[pl.ds(base0, _CH)], sets[1][5]).wait()

    return gather_k


@functools.lru_cache(maxsize=None)
def _segsum_fn(n_nodes, n_edges, lat):
    per_sc = n_edges // _NC
    per_w = per_sc // _NS
    nfull, tail = divmod(per_w, _CH)
    w = 2 * lat
    mesh = plsc.VectorSubcoreMesh(core_axis_name="c", subcore_axis_name="s")
    scratch = [pltpu.VMEM_SHARED((n_nodes, w), jnp.float32)]
    for _ in range(2):
        scratch += [pltpu.VMEM((_CH, w), jnp.float32),
                    pltpu.VMEM((_CH,), jnp.int32),
                    pltpu.SemaphoreType.DMA, pltpu.SemaphoreType.DMA,
                    pltpu.SemaphoreType.DMA]
    if tail:
        scratch += [pltpu.VMEM((tail, w), jnp.float32),
                    pltpu.VMEM((tail,), jnp.int32)]

    @functools.partial(
        pl.kernel,
        out_type=jax.ShapeDtypeStruct((_NC, n_nodes, w), jnp.float32),
        mesh=mesh, scratch_types=scratch)
    def segsum_k(e, dst, zeros, out, *refs):
        acc = refs[0]
        sets = [refs[1:6], refs[6:11]]
        if tail:
            tb, tj = refs[11], refs[12]
        cid = lax.axis_index("c")
        sid = lax.axis_index("s")

        @pl.when(sid == 0)
        def _():
            pltpu.sync_copy(zeros, acc)

        base0 = cid * per_sc + sid * per_w

        def start(i, s, first=False):
            bf, jx, se, si, ss = s
            if not first:
                pltpu.make_async_copy(bf, acc.at[jx], ss).wait()
            pltpu.async_copy(e.at[pl.ds(base0 + i * _CH, _CH)], bf, se)
            pltpu.async_copy(dst.at[pl.ds(base0 + i * _CH, _CH)], jx, si)

        def finish(i, s):
            bf, jx, se, si, ss = s
            pltpu.make_async_copy(
                e.at[pl.ds(base0 + i * _CH, _CH)], bf, se).wait()
            pltpu.make_async_copy(
                dst.at[pl.ds(base0 + i * _CH, _CH)], jx, si).wait()
            pltpu.async_copy(bf, acc.at[jx], ss, add=True)

        start(0, sets[0], first=True)
        if nfull > 1:
            start(1, sets[1], first=True)
        plsc.subcore_barrier()

        @pl.loop(0, nfull - nfull % 2, step=2)
        def _(i):
            finish(i, sets[0])

            @pl.when(i + 2 < nfull)
            def _():
                start(i + 2, sets[0])

            finish(i + 1, sets[1])

            @pl.when(i + 3 < nfull)
            def _():
                start(i + 3, sets[1])

        if nfull % 2:
            finish(nfull - 1, sets[0])

        if tail:
            base_t = base0 + nfull * _CH
            pltpu.async_copy(e.at[pl.ds(base_t, tail)], tb, sets[0][2])
            pltpu.async_copy(dst.at[pl.ds(base_t, tail)], tj, sets[0][3])
            pltpu.make_async_copy(e.at[pl.ds(base_t, tail)], tb, sets[0][2]).wait()
            pltpu.make_async_copy(dst.at[pl.ds(base_t, tail)], tj, sets[0][3]).wait()
            pltpu.sync_copy(tb, acc.at[tj], add=True)

        if nfull > 0:
            pltpu.make_async_copy(
                sets[0][0], acc.at[sets[0][1]], sets[0][4]).wait()
        if nfull > 1:
            pltpu.make_async_copy(
                sets[1][0], acc.at[sets[1][1]], sets[1][4]).wait()

        plsc.subcore_barrier()

        @pl.when(sid == 0)
        def _():
            pltpu.sync_copy(acc, out.at[cid])

    return segsum_k


def kernel(x, edge_index, edge_features, params):
    src = edge_index[0]
    dst = edge_index[1]
    n_nodes = x.shape[0]
    n_edges = src.shape[0]

    def v(t):
        return t.reshape(1, -1)

    blocks = params["blocks"]
    nblk = len(blocks)

    en = params["enc_node"]
    (wn1, bn1), (wn2, bn2) = en["layers"]
    gn, betan = en["ln"]
    lat = wn2.shape[1]

    def packed_w1(i):
        w1 = blocks[i]["edge"]["layers"][0][0]
        b1 = blocks[i]["edge"]["layers"][0][1]
        wab = jnp.concatenate([w1[:lat], w1[lat:2 * lat]], axis=1)
        bab = jnp.concatenate([b1, jnp.zeros_like(b1)]).reshape(1, -1)
        return wab, bab

    wab0, bab0 = packed_w1(0)
    h, hab = _enc_node(x, wn1, v(bn1), wn2, v(bn2), v(gn), v(betan),
                       wab0, bab0)

    ee = params["enc_edge"]
    (we1, be1), (we2, be2) = ee["layers"]
    ge, betae = ee["ln"]
    half = n_edges // 2
    eh = [_enc_edge(edge_features, we1, v(be1), we2, v(be2), v(ge), v(betae),
                    o, half) for o in (0, half)]

    src_h = [lax.slice_in_dim(src, 0, half), lax.slice_in_dim(src, half, n_edges)]
    dst_h = [lax.slice_in_dim(dst, 0, half), lax.slice_in_dim(dst, half, n_edges)]

    zeros = jnp.zeros((n_nodes, 2 * lat), jnp.float32)
    (wdn1, bdn1), (wdn2, bdn2) = params["dec_node"]["layers"]
    (wde1, bde1), (wde2, bde2) = params["dec_edge"]["layers"]

    gather = _gather_fn(n_nodes, half, lat)
    segsum = _segsum_fn(n_nodes, half, lat)

    node_out = edge_out = None
    for i in range(nblk):
        blk = blocks[i]
        (ew1, _), (ew2, eb2) = blk["edge"]["layers"]
        eg, ebeta = blk["edge"]["ln"]
        w1c = ew1[2 * lat:]

        parts = [None, None]
        dec_h = [None, None]
        for s in (0, 1):
            gsum = gather(hab, src_h[s], dst_h[s])
            if i < nblk - 1:
                eh[s] = _edge_blk(gsum, eh[s], w1c, ew2, v(eb2), v(eg),
                                  v(ebeta))
            else:
                eh[s], dec_h[s] = _edge_blk_dec(
                    gsum, eh[s], w1c, ew2, v(eb2), v(eg), v(ebeta),
                    wde1, v(bde1), wde2, v(bde2))
            parts[s] = segsum(eh[s], dst_h[s], zeros)

        (nw1, nb1), (nw2, nb2) = blk["node"]["layers"]
        ng, nbeta = blk["node"]["ln"]
        w1h, w1g = nw1[:lat], nw1[lat:]
        if i < nblk - 1:
            nwab, nbab = packed_w1(i + 1)
            h, hab = _node_blk(h, parts[0], parts[1], w1h, w1g, v(nb1), nw2,
                               v(nb2), v(ng), v(nbeta), nwab, nbab)
        else:
            node_out = _node_blk_dec(h, parts[0], parts[1], w1h, w1g, v(nb1),
                                     nw2, v(nb2), v(ng), v(nbeta),
                                     wdn1, v(bdn1), wdn2, v(bdn2))
            edge_out = jnp.concatenate(dec_h, axis=0)

    return (node_out, edge_out)
